# Initial kernel scaffold; baseline (speedup 1.0000x reference)
import jax
import jax.numpy as jnp
from jax.experimental import pallas as pl


def kernel(x, edge_index, W, bias):
    raise NotImplementedError("write your pallas kernel here")


# speedup vs baseline: 10.7767x; 10.7767x over previous
import functools

import jax
import jax.numpy as jnp
from jax import lax
from jax.experimental import pallas as pl
from jax.experimental.pallas import tpu as pltpu
from jax.experimental.pallas import tpu_sc as plsc

N = 10000
D = 128
E = 320000
NC, NS = 2, 16
NW = NC * NS
NP = 10240
ER = 2560
RT = ER // NW
NPT = NP // NS

_mesh = plsc.VectorSubcoreMesh(
    core_axis_name="c", subcore_axis_name="s", num_cores=NC, num_subcores=NS)


def _set_identity(iidx, base):
    for j in range(8):
        iidx[pl.ds(j * 16, 16)] = lax.iota(jnp.int32, 16) + (base + j * 16)


@functools.partial(
    pl.kernel,
    out_type=jax.ShapeDtypeStruct((NC * NP, D), jnp.float32),
    mesh=_mesh,
    scratch_types=[
        pltpu.VMEM((128,), jnp.int32),
        pltpu.VMEM((128,), jnp.int32),
        pltpu.VMEM((128, D), jnp.float32),
        pltpu.VMEM((128, D), jnp.float32),
        pltpu.VMEM_SHARED((NP, D), jnp.float32),
        pltpu.SemaphoreType.DMA,
    ],
)
def _deg_kernel(col_hbm, ones_hbm, zeros_hbm, deg_out, iidx, cidx, ones_v,
                stage, deg_sh, sem):
    c = lax.axis_index("c")
    s = lax.axis_index("s")
    b = c * NS + s

    pltpu.sync_copy(zeros_hbm, stage)
    pltpu.sync_copy(ones_hbm, ones_v)
    for k in range(NPT // 128):
        _set_identity(iidx, s * NPT + k * 128)
        pltpu.sync_copy(stage, deg_sh.at[iidx])
    plsc.subcore_barrier()

    def body(j, carry):
        pltpu.sync_copy(col_hbm.at[b * RT + j], cidx)
        pltpu.sync_copy(ones_v, deg_sh.at[cidx], add=True)
        return carry

    lax.fori_loop(0, RT, body, 0, unroll=False)
    plsc.subcore_barrier()

    for k in range(NPT // 128):
        _set_identity(iidx, s * NPT + k * 128)
        pltpu.async_copy(deg_sh.at[iidx], stage, sem).wait()
        pltpu.sync_copy(stage, deg_out.at[pl.ds(b * NPT + k * 128, 128)])


@functools.partial(
    pl.kernel,
    out_type=jax.ShapeDtypeStruct((NC * NP, D), jnp.float32),
    mesh=_mesh,
    scratch_types=[
        pltpu.VMEM((128,), jnp.int32),
        pltpu.VMEM((128,), jnp.int32),
        pltpu.VMEM((128,), jnp.int32),
        pltpu.VMEM((128, D), jnp.float32),
        pltpu.VMEM((128, D), jnp.float32),
        pltpu.VMEM_SHARED((NP, D), jnp.float32),
        pltpu.SemaphoreType.DMA,
    ],
)
def _edge_kernel(row_hbm, col_hbm, y_hbm, zeros_hbm, acc_out, iidx, ridx,
                 cidx, gbuf, stage, acc_sh, sem):
    c = lax.axis_index("c")
    s = lax.axis_index("s")
    b = c * NS + s

    pltpu.sync_copy(zeros_hbm, stage)
    for k in range(NPT // 128):
        _set_identity(iidx, s * NPT + k * 128)
        pltpu.sync_copy(stage, acc_sh.at[iidx])
    plsc.subcore_barrier()

    def body(j, carry):
        pltpu.sync_copy(row_hbm.at[b * RT + j], ridx)
        pltpu.sync_copy(col_hbm.at[b * RT + j], cidx)
        pltpu.async_copy(y_hbm.at[ridx], gbuf, sem).wait()
        pltpu.sync_copy(gbuf, acc_sh.at[cidx], add=True)
        return carry

    lax.fori_loop(0, RT, body, 0, unroll=False)
    plsc.subcore_barrier()

    for k in range(NPT // 128):
        _set_identity(iidx, s * NPT + k * 128)
        pltpu.async_copy(acc_sh.at[iidx], gbuf, sem).wait()
        pltpu.sync_copy(gbuf, acc_out.at[pl.ds(b * NPT + k * 128, 128)])


def _transform_body(x_ref, w_ref, d0_ref, d1_ref, y_ref):
    xw = lax.dot_general(x_ref[...], w_ref[...], (((1,), (1,)), ((), ())),
                         preferred_element_type=jnp.float32,
                         precision=lax.Precision.HIGHEST)
    deg = d0_ref[...] + d1_ref[...] + 1.0
    y_ref[...] = xw * lax.rsqrt(deg)


def _finalize_body(a0_ref, a1_ref, y_ref, d0_ref, d1_ref, b_ref, o_ref):
    deg = d0_ref[...] + d1_ref[...] + 1.0
    acc = a0_ref[...] + a1_ref[...] + y_ref[...]
    o_ref[...] = acc * lax.rsqrt(deg) + b_ref[...]


_BM = 2000


def _node_spec():
    return pl.BlockSpec((_BM, D), lambda i: (i, 0))


def kernel(x, edge_index, W, bias):
    ei = edge_index.astype(jnp.int32)
    pad = ER * 128 - E
    row = jnp.concatenate([ei[0], jnp.zeros((pad,), jnp.int32)])
    col = jnp.concatenate([ei[1], jnp.full((pad,), NP - 1, jnp.int32)])
    row2 = row.reshape(ER, 128)
    col2 = col.reshape(ER, 128)

    ones_s = jnp.ones((128, D), jnp.float32)
    zeros_s = jnp.zeros((128, D), jnp.float32)

    deg2 = _deg_kernel(col2, ones_s, zeros_s)
    d0 = deg2[:N]
    d1 = deg2[NP:NP + N]

    y = pl.pallas_call(
        _transform_body,
        grid=(N // _BM,),
        in_specs=[
            _node_spec(),
            pl.BlockSpec((D, D), lambda i: (0, 0)),
            _node_spec(),
            _node_spec(),
        ],
        out_specs=_node_spec(),
        out_shape=jax.ShapeDtypeStruct((N, D), jnp.float32),
    )(x, W, d0, d1)

    acc2 = _edge_kernel(row2, col2, y, zeros_s)
    a0 = acc2[:N]
    a1 = acc2[NP:NP + N]

    out = pl.pallas_call(
        _finalize_body,
        grid=(N // _BM,),
        in_specs=[
            _node_spec(),
            _node_spec(),
            _node_spec(),
            _node_spec(),
            _node_spec(),
            pl.BlockSpec((1, D), lambda i: (0, 0)),
        ],
        out_specs=_node_spec(),
        out_shape=jax.ShapeDtypeStruct((N, D), jnp.float32),
    )(a0, a1, y, d0, d1, bias.reshape(1, D))
    return out


# speedup vs baseline: 13.0555x; 1.2115x over previous
import functools

import jax
import jax.numpy as jnp
from jax import lax
from jax.experimental import pallas as pl
from jax.experimental.pallas import tpu as pltpu
from jax.experimental.pallas import tpu_sc as plsc

N = 10000
D = 128
E = 320000
NC, NS = 2, 16
NW = NC * NS
NP = 10240
ER = 2560
RT = ER // NW
NPT = NP // NS

_mesh = plsc.VectorSubcoreMesh(
    core_axis_name="c", subcore_axis_name="s", num_cores=NC, num_subcores=NS)


def _set_identity(iidx, base):
    for j in range(8):
        iidx[pl.ds(j * 16, 16)] = lax.iota(jnp.int32, 16) + (base + j * 16)


@functools.partial(
    pl.kernel,
    out_type=jax.ShapeDtypeStruct((NC * NP, D), jnp.float32),
    mesh=_mesh,
    scratch_types=[
        pltpu.VMEM((128,), jnp.int32),
        pltpu.VMEM((128,), jnp.int32),
        pltpu.VMEM((128, D), jnp.float32),
        pltpu.VMEM((128, D), jnp.float32),
        pltpu.VMEM_SHARED((NP, D), jnp.float32),
        pltpu.SemaphoreType.DMA,
    ],
)
def _deg_kernel(col_hbm, ones_hbm, zeros_hbm, deg_out, iidx, cidx, ones_v,
                stage, deg_sh, sem):
    c = lax.axis_index("c")
    s = lax.axis_index("s")
    b = c * NS + s

    pltpu.sync_copy(zeros_hbm, stage)
    pltpu.sync_copy(ones_hbm, ones_v)
    for k in range(NPT // 128):
        _set_identity(iidx, s * NPT + k * 128)
        pltpu.sync_copy(stage, deg_sh.at[iidx])
    plsc.subcore_barrier()

    def body(j, carry):
        pltpu.sync_copy(col_hbm.at[b * RT + j], cidx)
        pltpu.sync_copy(ones_v, deg_sh.at[cidx], add=True)
        return carry

    lax.fori_loop(0, RT, body, 0, unroll=False)
    plsc.subcore_barrier()

    for k in range(NPT // 128):
        _set_identity(iidx, s * NPT + k * 128)
        pltpu.async_copy(deg_sh.at[iidx], stage, sem).wait()
        pltpu.sync_copy(stage, deg_out.at[pl.ds(b * NPT + k * 128, 128)])


@functools.partial(
    pl.kernel,
    out_type=jax.ShapeDtypeStruct((NC * NP, D), jnp.float32),
    mesh=_mesh,
    scratch_types=[
        pltpu.VMEM((128,), jnp.int32),
        [pltpu.VMEM((128,), jnp.int32) for _ in range(4)],
        [pltpu.VMEM((128,), jnp.int32) for _ in range(4)],
        [pltpu.VMEM((128, D), jnp.float32) for _ in range(2)],
        pltpu.VMEM_SHARED((NP, D), jnp.float32),
        pltpu.SemaphoreType.DMA,
        [pltpu.SemaphoreType.DMA for _ in range(4)],
        [pltpu.SemaphoreType.DMA for _ in range(2)],
        [pltpu.SemaphoreType.DMA for _ in range(2)],
    ],
)
def _edge_kernel(row_hbm, col_hbm, y_hbm, zeros_hbm, acc_out, iidx, ridx,
                 cidx, gbuf, acc_sh, sem, isem, gsem, ssem):
    c = lax.axis_index("c")
    s = lax.axis_index("s")
    b = c * NS + s

    pltpu.sync_copy(zeros_hbm, gbuf[0])
    for k in range(NPT // 128):
        _set_identity(iidx, s * NPT + k * 128)
        pltpu.sync_copy(gbuf[0], acc_sh.at[iidx])
    plsc.subcore_barrier()

    def start_idx(j, r):
        pltpu.async_copy(row_hbm.at[b * RT + j], ridx[r], isem[r])
        pltpu.async_copy(col_hbm.at[b * RT + j], cidx[r], isem[r])

    def wait_idx(r):
        pltpu.make_async_copy(row_hbm.at[0], ridx[r], isem[r]).wait()
        pltpu.make_async_copy(col_hbm.at[0], cidx[r], isem[r]).wait()

    def start_gather(r, p):
        pltpu.async_copy(y_hbm.at[ridx[r]], gbuf[p], gsem[p])

    def wait_gather(p):
        pltpu.make_async_copy(y_hbm.at[pl.ds(0, 128)], gbuf[p],
                              gsem[p]).wait()

    def start_scatter(r, p):
        pltpu.async_copy(gbuf[p], acc_sh.at[cidx[r]], ssem[p], add=True)

    def wait_scatter(p):
        pltpu.make_async_copy(y_hbm.at[pl.ds(0, 128)], gbuf[p],
                              ssem[p]).wait()

    def phase(jv, k, do_prev=True, do_g=True, do_i=True):
        p, q = k % 2, 1 - (k % 2)
        if do_prev:
            wait_scatter(q)
        if do_g:
            wait_idx((k + 1) % 4)
            start_gather((k + 1) % 4, q)
        if do_i:
            start_idx(jv + 3, (k + 3) % 4)
        wait_gather(p)
        start_scatter(k, p)

    for r in range(3):
        start_idx(r, r)
    wait_idx(0)
    start_gather(0, 0)

    phase(0, 0, do_prev=False)
    phase(1, 1)
    phase(2, 2)
    phase(3, 3)

    def body(t, carry):
        for k in range(4):
            phase(4 * t + k, k)
        return carry

    lax.fori_loop(1, RT // 4 - 1, body, 0, unroll=False)

    phase(RT - 4, 0)
    phase(RT - 3, 1, do_i=False)
    phase(RT - 2, 2, do_i=False)
    phase(RT - 1, 3, do_g=False, do_i=False)
    wait_scatter((RT - 1) % 2)
    plsc.subcore_barrier()

    for k in range(NPT // 128):
        _set_identity(iidx, s * NPT + k * 128)
        pltpu.async_copy(acc_sh.at[iidx], gbuf[0], sem).wait()
        pltpu.sync_copy(gbuf[0], acc_out.at[pl.ds(b * NPT + k * 128, 128)])


def _transform_body(x_ref, w_ref, d0_ref, d1_ref, y_ref):
    xw = lax.dot_general(x_ref[...], w_ref[...], (((1,), (1,)), ((), ())),
                         preferred_element_type=jnp.float32,
                         precision=lax.Precision.HIGHEST)
    deg = d0_ref[...] + d1_ref[...] + 1.0
    y_ref[...] = xw * lax.rsqrt(deg)


def _finalize_body(a0_ref, a1_ref, y_ref, d0_ref, d1_ref, b_ref, o_ref):
    deg = d0_ref[...] + d1_ref[...] + 1.0
    acc = a0_ref[...] + a1_ref[...] + y_ref[...]
    o_ref[...] = acc * lax.rsqrt(deg) + b_ref[...]


_BM = 2000


def _node_spec():
    return pl.BlockSpec((_BM, D), lambda i: (i, 0))


def kernel(x, edge_index, W, bias):
    ei = edge_index.astype(jnp.int32)
    pad = ER * 128 - E
    row = jnp.concatenate([ei[0], jnp.zeros((pad,), jnp.int32)])
    col = jnp.concatenate([ei[1], jnp.full((pad,), NP - 1, jnp.int32)])
    row2 = row.reshape(ER, 128)
    col2 = col.reshape(ER, 128)

    ones_s = jnp.ones((128, D), jnp.float32)
    zeros_s = jnp.zeros((128, D), jnp.float32)

    deg2 = _deg_kernel(col2, ones_s, zeros_s)
    d0 = deg2[:N]
    d1 = deg2[NP:NP + N]

    y = pl.pallas_call(
        _transform_body,
        grid=(N // _BM,),
        in_specs=[
            _node_spec(),
            pl.BlockSpec((D, D), lambda i: (0, 0)),
            _node_spec(),
            _node_spec(),
        ],
        out_specs=_node_spec(),
        out_shape=jax.ShapeDtypeStruct((N, D), jnp.float32),
    )(x, W, d0, d1)

    acc2 = _edge_kernel(row2, col2, y, zeros_s)
    a0 = acc2[:N]
    a1 = acc2[NP:NP + N]

    out = pl.pallas_call(
        _finalize_body,
        grid=(N // _BM,),
        in_specs=[
            _node_spec(),
            _node_spec(),
            _node_spec(),
            _node_spec(),
            _node_spec(),
            pl.BlockSpec((1, D), lambda i: (0, 0)),
        ],
        out_specs=_node_spec(),
        out_shape=jax.ShapeDtypeStruct((N, D), jnp.float32),
    )(a0, a1, y, d0, d1, bias.reshape(1, D))
    return out
